# baseline (speedup 1.0000x reference)
import functools

import jax
import jax.numpy as jnp
from jax import lax
from jax.experimental import pallas as pl
from jax.experimental.pallas import tpu as pltpu
from jax.experimental.pallas import tpu_sc as plsc

VOCAB = 1000
EMBED_DIM = 64
BATCH = 4096
SEQ = 200

N = BATCH * SEQ
NUM_CORES = 2
NUM_SUBCORES = 16
NW = NUM_CORES * NUM_SUBCORES
PER_W = N // NW
IDX_MINOR = 128
CHUNK = 512
IDX_ROWS = CHUNK // IDX_MINOR
NCHUNKS = PER_W // CHUNK

_mesh = plsc.VectorSubcoreMesh(core_axis_name="c", subcore_axis_name="s")


@functools.partial(
    pl.kernel,
    mesh=_mesh,
    out_type=jax.ShapeDtypeStruct((N, EMBED_DIM), jnp.float32),
    scratch_types=[
        pltpu.VMEM((IDX_ROWS, IDX_MINOR), jnp.int32),
        pltpu.VMEM((IDX_ROWS, IDX_MINOR), jnp.int32),
        pltpu.VMEM((CHUNK, EMBED_DIM), jnp.float32),
        pltpu.VMEM((CHUNK, EMBED_DIM), jnp.float32),
        pltpu.SemaphoreType.DMA,
        pltpu.SemaphoreType.DMA,
        pltpu.SemaphoreType.DMA,
        pltpu.SemaphoreType.DMA,
    ],
    compiler_params=pltpu.CompilerParams(use_tc_tiling_on_sc=False),
)
def _sc_gather(idx_hbm, table_hbm, out_hbm,
               idx0, idx1, rows0, rows1, g0, g1, w0, w1):
    wid = lax.axis_index("s") * NUM_CORES + lax.axis_index("c")
    idx_row0 = wid * (PER_W // IDX_MINOR)
    out_row0 = wid * PER_W
    idxs, rows, gsems, wsems = (idx0, idx1), (rows0, rows1), (g0, g1), (w0, w1)

    def idxload(ci, b):
        pltpu.sync_copy(idx_hbm.at[pl.ds(idx_row0 + ci * IDX_ROWS, IDX_ROWS)],
                        idxs[b])

    def fire_gather(b):
        for j in range(IDX_ROWS):
            pltpu.async_copy(table_hbm.at[idxs[b].at[j]],
                             rows[b].at[pl.ds(j * IDX_MINOR, IDX_MINOR)],
                             gsems[b])

    def wait_gather(b):
        pltpu.make_async_copy(out_hbm.at[pl.ds(0, CHUNK)], rows[b],
                              gsems[b]).wait()

    def fire_wb(ci, b):
        pltpu.async_copy(rows[b],
                         out_hbm.at[pl.ds(out_row0 + ci * CHUNK, CHUNK)],
                         wsems[b])

    def wait_wb(b):
        pltpu.make_async_copy(rows[b], out_hbm.at[pl.ds(0, CHUNK)],
                              wsems[b]).wait()

    for b in range(2):
        idxload(b, b)
        fire_gather(b)

    def body(i, carry):
        for b in range(2):
            ci = 2 * i + b
            wait_gather(b)
            fire_wb(ci, b)
            idxload(ci + 2, b)
            wait_wb(b)
            fire_gather(b)
        return carry

    lax.fori_loop(0, NCHUNKS // 2 - 1, body, 0)

    for b in range(2):
        wait_gather(b)
        fire_wb(NCHUNKS - 2 + b, b)
        wait_wb(b)


def kernel(word_sequences, table):
    idx = word_sequences.reshape(N // IDX_MINOR, IDX_MINOR).astype(jnp.int32)
    out = _sc_gather(idx, table)
    return out.reshape(BATCH, SEQ, EMBED_DIM)


# speedup vs baseline: 1.0683x; 1.0683x over previous
import functools

import jax
import jax.numpy as jnp
from jax import lax
from jax.experimental import pallas as pl
from jax.experimental.pallas import tpu as pltpu
from jax.experimental.pallas import tpu_sc as plsc

VOCAB = 1000
EMBED_DIM = 64
BATCH = 4096
SEQ = 200

NUM_CORES = 2
NUM_SUBCORES = 16
NW = NUM_CORES * NUM_SUBCORES
BCHUNK = 256
BUFW = BCHUNK + 1
NBB = BATCH // BCHUNK
NBLOCKS = SEQ * NBB
PER_W = NBLOCKS // NW
LANES = 16
NJ = EMBED_DIM // LANES

_mesh = plsc.VectorSubcoreMesh(core_axis_name="c", subcore_axis_name="s")


@functools.partial(
    pl.kernel,
    mesh=_mesh,
    out_type=jax.ShapeDtypeStruct((SEQ, EMBED_DIM, BATCH), jnp.float32),
    scratch_types=[
        pltpu.VMEM((VOCAB * EMBED_DIM,), jnp.float32),
        pltpu.VMEM((2, 1, BCHUNK), jnp.int32),
        pltpu.VMEM((2, 1, EMBED_DIM, BUFW), jnp.float32),
        pltpu.SemaphoreType.DMA,
        pltpu.SemaphoreType.DMA,
        pltpu.SemaphoreType.DMA,
    ],
    compiler_params=pltpu.CompilerParams(use_tc_tiling_on_sc=True,
                                         needs_layout_passes=False),
)
def _sc_lookup(idx_hbm, table_hbm, out_hbm,
               table_v, idx_v, out_v, isem, wsem_a, wsem_b):
    wid = lax.axis_index("s") * NUM_CORES + lax.axis_index("c")
    h0 = wid * PER_W
    wsems = (wsem_a, wsem_b)
    iota = lax.iota(jnp.int32, LANES)
    rows = [iota + j * LANES for j in range(NJ)]
    zeros16 = jnp.zeros((LANES,), jnp.int32)

    pltpu.sync_copy(table_hbm, table_v)

    def idx_window(h):
        s = lax.shift_right_logical(h, 4)
        bb = lax.bitwise_and(h, NBB - 1)
        return idx_hbm.at[pl.ds(s, 1),
                          pl.ds(pl.multiple_of(bb * BCHUNK, BCHUNK), BCHUNK)]

    def out_window(h):
        s = lax.shift_right_logical(h, 4)
        bb = lax.bitwise_and(h, NBB - 1)
        return out_hbm.at[pl.ds(s, 1), pl.ds(0, EMBED_DIM),
                          pl.ds(pl.multiple_of(bb * BCHUNK, BCHUNK), BCHUNK)]

    pltpu.sync_copy(idx_window(h0), idx_v.at[0])

    def body(t, carry):
        for q in range(2):
            i = 2 * t + q
            h = h0 + i

            @pl.when(i > 0)
            def _():
                pltpu.make_async_copy(idx_window(h), idx_v.at[q], isem).wait()

            @pl.when(i < PER_W - 1)
            def _():
                pltpu.async_copy(idx_window(h + 1), idx_v.at[1 - q], isem)

            @pl.when(t > 0)
            def _():
                pltpu.make_async_copy(out_v.at[q, :, :, pl.ds(0, BCHUNK)],
                                      out_window(h0), wsems[q]).wait()

            @plsc.parallel_loop(0, BCHUNK // LANES, unroll=2)
            def fill(bg):
                b0 = bg * LANES
                idx16 = idx_v[q, 0, pl.ds(b0, LANES)]
                for l in range(LANES):
                    base = idx16[l] * EMBED_DIM
                    colb = zeros16 + (b0 + l)
                    for j in range(NJ):
                        val = table_v[pl.ds(base + j * LANES, LANES)]
                        plsc.store_scatter(out_v.at[q, 0], [rows[j], colb],
                                           val)

            pltpu.async_copy(out_v.at[q, :, :, pl.ds(0, BCHUNK)],
                             out_window(h), wsems[q])
        return carry

    lax.fori_loop(0, PER_W // 2, body, 0)

    for q in range(2):
        pltpu.make_async_copy(out_v.at[q, :, :, pl.ds(0, BCHUNK)],
                              out_window(h0), wsems[q]).wait()


def kernel(word_sequences, table):
    idx_t = word_sequences.astype(jnp.int32).T
    table_flat = table.reshape(VOCAB * EMBED_DIM)
    out_t = _sc_lookup(idx_t, table_flat)
    return out_t.transpose(2, 0, 1)
